# baseline (speedup 1.0000x reference)
import functools

import jax
import jax.numpy as jnp
from jax import lax
from jax.experimental import pallas as pl
from jax.experimental.pallas import tpu as pltpu
from jax.experimental.pallas import tpu_sc as plsc

B = 16384
D = 128
NC = 2
NS = 16
NW = NC * NS
LANES = 16
ROWS_PER_W = B // NW
CHUNK_ROWS = 128
CHUNKS = ROWS_PER_W // CHUNK_ROWS

_mesh = plsc.VectorSubcoreMesh(core_axis_name="c", subcore_axis_name="s")


@functools.partial(
    pl.kernel,
    mesh=_mesh,
    out_type=jax.ShapeDtypeStruct((B, D), jnp.float32),
    scratch_types=[
        pltpu.VMEM((1,), jnp.int32),
        pltpu.VMEM((1, D), jnp.float32),
        pltpu.VMEM((1, D), jnp.float32),
        pltpu.VMEM((CHUNK_ROWS, D), jnp.float32),
        pltpu.SemaphoreType.DMA,
    ],
)
def _sep_bias_sc(in_hbm, lab_hbm, scale_hbm, off_hbm, out_hbm,
                 idx_v, srow_v, orow_v, buf_v, sem):
    wid = lax.axis_index("s") * NC + lax.axis_index("c")
    base = wid * ROWS_PER_W

    pltpu.sync_copy(lab_hbm, idx_v)
    pltpu.async_copy(scale_hbm.at[idx_v], srow_v, sem).wait()
    pltpu.async_copy(off_hbm.at[idx_v], orow_v, sem).wait()

    svec = [srow_v[0, pl.ds(LANES * j, LANES)] for j in range(D // LANES)]
    ovec = [orow_v[0, pl.ds(LANES * j, LANES)] for j in range(D // LANES)]

    for c in range(CHUNKS):
        r0 = base + c * CHUNK_ROWS
        pltpu.sync_copy(in_hbm.at[pl.ds(r0, CHUNK_ROWS)], buf_v)

        def row_body(r, carry):
            for j in range(D // LANES):
                x = buf_v[r, pl.ds(LANES * j, LANES)]
                buf_v[r, pl.ds(LANES * j, LANES)] = jnp.maximum(
                    x * svec[j] + ovec[j], 0.0)
            return carry

        lax.fori_loop(0, CHUNK_ROWS, row_body, jnp.int32(0))
        pltpu.sync_copy(buf_v, out_hbm.at[pl.ds(r0, CHUNK_ROWS)])


def kernel(inputs, label, scale_table, offset_table):
    lab = jnp.asarray(label, jnp.int32).reshape(1)
    return _sep_bias_sc(inputs, lab, scale_table, offset_table)


# speedup vs baseline: 1.1978x; 1.1978x over previous
import functools

import jax
import jax.numpy as jnp
from jax import lax
from jax.experimental import pallas as pl
from jax.experimental.pallas import tpu as pltpu
from jax.experimental.pallas import tpu_sc as plsc

B = 16384
D = 128
NC = 2
NS = 16
NW = NC * NS
LANES = 16
ROWS_PER_W = B // NW
CHUNK_ROWS = 128
CHUNKS = ROWS_PER_W // CHUNK_ROWS
NBUF = 2

_mesh = plsc.VectorSubcoreMesh(core_axis_name="c", subcore_axis_name="s")


@functools.partial(
    pl.kernel,
    mesh=_mesh,
    out_type=jax.ShapeDtypeStruct((B, D), jnp.float32),
    scratch_types=[
        pltpu.VMEM((1,), jnp.int32),
        pltpu.VMEM((1, D), jnp.float32),
        pltpu.VMEM((1, D), jnp.float32),
        pltpu.VMEM((NBUF, CHUNK_ROWS, D), jnp.float32),
        pltpu.VMEM((NBUF, CHUNK_ROWS, D), jnp.float32),
        pltpu.SemaphoreType.DMA,
        pltpu.SemaphoreType.DMA,
        pltpu.SemaphoreType.DMA,
        pltpu.SemaphoreType.DMA,
        pltpu.SemaphoreType.DMA,
    ],
)
def _sep_bias_sc(in_hbm, lab_hbm, scale_hbm, off_hbm, out_hbm,
                 idx_v, srow_v, orow_v, ibufs, obufs,
                 gsem, si0, si1, so0, so1):
    wid = lax.axis_index("s") * NC + lax.axis_index("c")
    base = wid * ROWS_PER_W
    si = [si0, si1]
    so = [so0, so1]

    h_in = {}
    for c in range(NBUF):
        h_in[c] = pltpu.async_copy(
            in_hbm.at[pl.ds(base + c * CHUNK_ROWS, CHUNK_ROWS)],
            ibufs.at[c % NBUF], si[c % NBUF])

    pltpu.sync_copy(lab_hbm, idx_v)
    pltpu.async_copy(scale_hbm.at[idx_v], srow_v, gsem).wait()
    pltpu.async_copy(off_hbm.at[idx_v], orow_v, gsem).wait()

    svec = [srow_v[0, pl.ds(LANES * j, LANES)] for j in range(D // LANES)]
    ovec = [orow_v[0, pl.ds(LANES * j, LANES)] for j in range(D // LANES)]

    h_out = {}
    for c in range(CHUNKS):
        b = c % NBUF
        h_in[c].wait()
        if c >= NBUF:
            h_out[c - NBUF].wait()
        ibuf = ibufs.at[b]
        obuf = obufs.at[b]

        def row_body(r, carry, ibuf=ibuf, obuf=obuf):
            for j in range(D // LANES):
                x = ibuf[r, pl.ds(LANES * j, LANES)]
                obuf[r, pl.ds(LANES * j, LANES)] = jnp.maximum(
                    x * svec[j] + ovec[j], 0.0)
            return carry

        lax.fori_loop(0, CHUNK_ROWS, row_body, jnp.int32(0))

        h_out[c] = pltpu.async_copy(
            obuf, out_hbm.at[pl.ds(base + c * CHUNK_ROWS, CHUNK_ROWS)],
            so[b])
        if c + NBUF < CHUNKS:
            h_in[c + NBUF] = pltpu.async_copy(
                in_hbm.at[pl.ds(base + (c + NBUF) * CHUNK_ROWS, CHUNK_ROWS)],
                ibufs.at[b], si[b])

    for c in range(CHUNKS - NBUF, CHUNKS):
        h_out[c].wait()


def kernel(inputs, label, scale_table, offset_table):
    lab = jnp.asarray(label, jnp.int32).reshape(1)
    return _sep_bias_sc(inputs, lab, scale_table, offset_table)
